# Initial kernel scaffold; baseline (speedup 1.0000x reference)
import jax
import jax.numpy as jnp
from jax.experimental import pallas as pl


def kernel(nf, ef, edge_index, n_iters, W_e, b_e, W_a, b_a, W_n, b_n, W_no, b_no, W_eo, b_eo):
    raise NotImplementedError("write your pallas kernel here")


# speedup vs baseline: 4.1150x; 4.1150x over previous
import functools

import jax
import jax.numpy as jnp
from jax import lax
from jax.experimental import pallas as pl
from jax.experimental.pallas import tpu as pltpu
from jax.experimental.pallas import tpu_sc as plsc

N = 10000
E = 320000
NF_DIM = 128
HNF = 128
HEF = 64
YW = 80

NC_ = 2
NS_ = 16
NW = NC_ * NS_
EW = E // NW
CH = 128
NFULL = EW // CH
TAIL = EW - NFULL * CH
NSTRIPE = N // NS_

_mesh = plsc.VectorSubcoreMesh(core_axis_name="c", subcore_axis_name="s")


@functools.partial(
    pl.kernel,
    out_type=(
        jax.ShapeDtypeStruct((E, HEF), jnp.float32),
        jax.ShapeDtypeStruct((E, HEF), jnp.float32),
    ),
    mesh=_mesh,
    scratch_types=(
        pltpu.VMEM((CH,), jnp.int32),
        pltpu.VMEM((CH,), jnp.int32),
        pltpu.VMEM((TAIL,), jnp.int32),
        pltpu.VMEM((TAIL,), jnp.int32),
        pltpu.VMEM((CH, HEF), jnp.float32),
        pltpu.VMEM((CH, HEF), jnp.float32),
        pltpu.VMEM((TAIL, HEF), jnp.float32),
        pltpu.VMEM((TAIL, HEF), jnp.float32),
        pltpu.SemaphoreType.DMA,
        pltpu.SemaphoreType.DMA,
    ),
    compiler_params=pltpu.CompilerParams(use_tc_tiling_on_sc=False),
)
def _sc_gather(p_hbm, q_hbm, src_hbm, dst_hbm, gs_hbm, gd_hbm,
               idx_s, idx_d, idx_s_t, idx_d_t, rows_s, rows_d,
               rows_s_t, rows_d_t, sem_s, sem_d):
    c = lax.axis_index("c")
    s = lax.axis_index("s")
    base0 = (c * NS_ + s) * EW

    @pl.loop(0, NFULL)
    def _chunk(i):
        base = base0 + i * CH
        pltpu.sync_copy(src_hbm.at[pl.ds(base, CH)], idx_s)
        pltpu.sync_copy(dst_hbm.at[pl.ds(base, CH)], idx_d)
        cs = pltpu.async_copy(p_hbm.at[idx_s], rows_s, sem_s)
        cd = pltpu.async_copy(q_hbm.at[idx_d], rows_d, sem_d)
        cs.wait()
        cd.wait()
        pltpu.sync_copy(rows_s, gs_hbm.at[pl.ds(base, CH)])
        pltpu.sync_copy(rows_d, gd_hbm.at[pl.ds(base, CH)])

    base = base0 + NFULL * CH
    pltpu.sync_copy(src_hbm.at[pl.ds(base, TAIL)], idx_s_t)
    pltpu.sync_copy(dst_hbm.at[pl.ds(base, TAIL)], idx_d_t)
    cs = pltpu.async_copy(p_hbm.at[idx_s_t], rows_s_t, sem_s)
    cd = pltpu.async_copy(q_hbm.at[idx_d_t], rows_d_t, sem_d)
    cs.wait()
    cd.wait()
    pltpu.sync_copy(rows_s_t, gs_hbm.at[pl.ds(base, TAIL)])
    pltpu.sync_copy(rows_d_t, gd_hbm.at[pl.ds(base, TAIL)])


@functools.partial(
    pl.kernel,
    out_type=jax.ShapeDtypeStruct((NC_, N, YW), jnp.float32),
    mesh=_mesh,
    scratch_types=(
        pltpu.VMEM((CH,), jnp.int32),
        pltpu.VMEM((TAIL,), jnp.int32),
        pltpu.VMEM((CH, YW), jnp.float32),
        pltpu.VMEM((TAIL, YW), jnp.float32),
        pltpu.VMEM_SHARED((N, YW), jnp.float32),
    ),
    compiler_params=pltpu.CompilerParams(use_tc_tiling_on_sc=False),
)
def _sc_scatter(y_hbm, dst_hbm, zer_hbm, z_hbm,
                idx, idx_t, ybuf, ybuf_t, table):
    c = lax.axis_index("c")
    s = lax.axis_index("s")
    base0 = (c * NS_ + s) * EW
    stripe = pl.ds(s * NSTRIPE, NSTRIPE)
    pltpu.sync_copy(zer_hbm.at[stripe], table.at[stripe])
    plsc.subcore_barrier()

    @pl.loop(0, NFULL)
    def _chunk(i):
        base = base0 + i * CH
        pltpu.sync_copy(dst_hbm.at[pl.ds(base, CH)], idx)
        pltpu.sync_copy(y_hbm.at[pl.ds(base, CH)], ybuf)
        pltpu.sync_copy(ybuf, table.at[idx], add=True)

    base = base0 + NFULL * CH
    pltpu.sync_copy(dst_hbm.at[pl.ds(base, TAIL)], idx_t)
    pltpu.sync_copy(y_hbm.at[pl.ds(base, TAIL)], ybuf_t)
    pltpu.sync_copy(ybuf_t, table.at[idx_t], add=True)

    plsc.subcore_barrier()
    pltpu.sync_copy(table.at[stripe], z_hbm.at[c, stripe])


def _matmul_call(f, n_rows, block_rows, n_in, n_out, extra_specs, out_shapes):
    grid = (n_rows // block_rows,)
    return pl.pallas_call(
        f,
        grid=grid,
        in_specs=[pl.BlockSpec((block_rows, n_in), lambda i: (i, 0))] + extra_specs,
        out_specs=[pl.BlockSpec((block_rows, s[1]), lambda i: (i, 0))
                   for s in out_shapes],
        out_shape=[jax.ShapeDtypeStruct(s, jnp.float32) for s in out_shapes],
    )


def _full(shape):
    return pl.BlockSpec(shape, lambda i: (0, 0))


BN = 1000
BE = 4000


def _node_pre_k(nf, ws, wd, wn, bn, pn_o, qn_o, nc_o):
    x = nf[...]
    pn_o[...] = jnp.dot(x, ws[...], preferred_element_type=jnp.float32)
    qn_o[...] = jnp.dot(x, wd[...], preferred_element_type=jnp.float32)
    nc_o[...] = jnp.dot(x, wn[...], preferred_element_type=jnp.float32) + bn[...]


def _edge_pre_k(ef, we, be, ec_o):
    ec_o[...] = jnp.dot(ef[...], we[...], preferred_element_type=jnp.float32) + be[...]


def _edge_k(gs, gd, r, wa, ba, we, ec, hef_o, y_o, r2_o):
    pre = gs[...] + gd[...] + r[...]
    hef = jnp.maximum(pre, 0.0)
    hef_o[...] = hef
    logit = jnp.sum(hef * wa[...], axis=1, keepdims=True) + ba[...]
    ex = jnp.exp(logit)
    y_o[...] = jnp.concatenate(
        [hef * ex, ex, jnp.zeros((hef.shape[0], YW - HEF - 1), jnp.float32)],
        axis=1)
    r2_o[...] = jnp.dot(hef, we[...], preferred_element_type=jnp.float32) + ec[...]


def _node_k(z0, z1, hnf, nc, wnh, wna, ws, wd, pn, qn, hnf_o, p_o, q_o):
    z = z0[...] + z1[...]
    agg = z[:, :HEF] / (z[:, HEF:HEF + 1] + 1e-16)
    h = (jnp.dot(hnf[...], wnh[...], preferred_element_type=jnp.float32)
         + jnp.dot(agg, wna[...], preferred_element_type=jnp.float32)
         + nc[...])
    h = jnp.maximum(h, 0.0)
    hnf_o[...] = h
    p_o[...] = jnp.dot(h, ws[...], preferred_element_type=jnp.float32) + pn[...]
    q_o[...] = jnp.dot(h, wd[...], preferred_element_type=jnp.float32) + qn[...]


def _readout_k(x, w, b, o):
    o[...] = jnp.dot(x[...], w[...], preferred_element_type=jnp.float32) + b[...]


def kernel(nf, ef, edge_index, n_iters, W_e, b_e, W_a, b_a, W_n, b_n,
           W_no, b_no, W_eo, b_eo):
    f32 = jnp.float32
    src = edge_index[0]
    dst = edge_index[1]

    We_s1 = W_e[0:128]
    We_s2 = W_e[128:256]
    We_d1 = W_e[256:384]
    We_d2 = W_e[384:512]
    We_e1 = W_e[512:576]
    We_e2 = W_e[576:592]
    Wn_h = W_n[0:128]
    Wn_nf = W_n[128:256]
    Wn_a = W_n[256:320]
    be = b_e.reshape(1, HEF)
    bn = b_n.reshape(1, HNF)
    wa = W_a.reshape(1, HEF)
    ba = b_a.reshape(1, 1)
    bno = b_no.reshape(1, 128)
    beo = b_eo.reshape(1, HEF)

    pn, qn, nc = _matmul_call(
        _node_pre_k, N, BN, NF_DIM, None,
        [_full((NF_DIM, HEF)), _full((NF_DIM, HEF)), _full((NF_DIM, HNF)),
         _full((1, HNF))],
        [(N, HEF), (N, HEF), (N, HNF)],
    )(nf, We_s2, We_d2, Wn_nf, bn)

    (ec,) = _matmul_call(
        _edge_pre_k, E, BE, 16, None,
        [_full((16, HEF)), _full((1, HEF))],
        [(E, HEF)],
    )(ef, We_e2, be)

    zer = jnp.zeros((N, YW), f32)
    hnf0 = jnp.zeros((N, HNF), f32)
    hef0 = jnp.zeros((E, HEF), f32)

    edge_call = _matmul_call(
        _edge_k, E, BE, HEF, None,
        [pl.BlockSpec((BE, HEF), lambda i: (i, 0)),
         pl.BlockSpec((BE, HEF), lambda i: (i, 0)),
         _full((1, HEF)), _full((1, 1)), _full((HEF, HEF)),
         pl.BlockSpec((BE, HEF), lambda i: (i, 0))],
        [(E, HEF), (E, YW), (E, HEF)],
    )

    node_call = _matmul_call(
        _node_k, N, BN, YW, None,
        [pl.BlockSpec((BN, YW), lambda i: (i, 0)),
         pl.BlockSpec((BN, HNF), lambda i: (i, 0)),
         pl.BlockSpec((BN, HNF), lambda i: (i, 0)),
         _full((HNF, HNF)), _full((HEF, HNF)),
         _full((HNF, HEF)), _full((HNF, HEF)),
         pl.BlockSpec((BN, HEF), lambda i: (i, 0)),
         pl.BlockSpec((BN, HEF), lambda i: (i, 0))],
        [(N, HNF), (N, HEF), (N, HEF)],
    )

    def body(_, carry):
        hnf, hef, p, q, r = carry
        gs, gd = _sc_gather(p, q, src, dst)
        hef2, y, r2 = edge_call(gs, gd, r, wa, ba, We_e1, ec)
        z = _sc_scatter(y, dst, zer)
        hnf2, p2, q2 = node_call(z[0], z[1], hnf, nc, Wn_h, Wn_a,
                                 We_s1, We_d1, pn, qn)
        return (hnf2, hef2, p2, q2, r2)

    hnf, hef, _, _, _ = lax.fori_loop(
        0, n_iters, body, (hnf0, hef0, pn, qn, ec))

    (unf,) = _matmul_call(
        _readout_k, N, BN, HNF, None,
        [_full((HNF, 128)), _full((1, 128))],
        [(N, 128)],
    )(hnf, W_no, bno)

    (uef,) = _matmul_call(
        _readout_k, E, BE, HEF, None,
        [_full((HEF, HEF)), _full((1, HEF))],
        [(E, HEF)],
    )(hef, W_eo, beo)

    return (unf, uef)


# speedup vs baseline: 4.9918x; 1.2131x over previous
import functools

import jax
import jax.numpy as jnp
from jax import lax
from jax.experimental import pallas as pl
from jax.experimental.pallas import tpu as pltpu
from jax.experimental.pallas import tpu_sc as plsc

N = 10000
E = 320000
NF_DIM = 128
HNF = 128
HEF = 64
YW = 80

NC_ = 2
NS_ = 16
NW = NC_ * NS_
EW = E // NW
GCH = 400
NGCH = EW // GCH
CH = 80
NIDX = EW // CH
G = 5
NCHUNK = NIDX // G
NPAIR = (NCHUNK - 1) // 2
NSTRIPE = N // NS_

_mesh = plsc.VectorSubcoreMesh(core_axis_name="c", subcore_axis_name="s")
_sc_params = pltpu.CompilerParams(use_tc_tiling_on_sc=False)


@functools.partial(
    pl.kernel,
    out_type=(
        jax.ShapeDtypeStruct((E, HEF), jnp.float32),
        jax.ShapeDtypeStruct((E, HEF), jnp.float32),
    ),
    mesh=_mesh,
    scratch_types=(
        pltpu.VMEM((NGCH, GCH), jnp.int32),
        pltpu.VMEM((NGCH, GCH), jnp.int32),
        pltpu.VMEM((2, GCH, HEF), jnp.float32),
        pltpu.SemaphoreType.DMA,
        pltpu.SemaphoreType.DMA,
        pltpu.SemaphoreType.DMA,
        pltpu.SemaphoreType.DMA,
    ),
    compiler_params=_sc_params,
)
def _sc_gather(p_hbm, q_hbm, src_hbm, dst_hbm, gs_hbm, gd_hbm,
               sidx, didx, rows, g0, g1, w0, w1):
    c = lax.axis_index("c")
    s = lax.axis_index("s")
    w = c * NS_ + s
    base0 = w * EW
    pltpu.sync_copy(src_hbm.at[pl.ds(w * NGCH, NGCH)], sidx)
    pltpu.sync_copy(dst_hbm.at[pl.ds(w * NGCH, NGCH)], didx)

    gsem = (g0, g1)
    wsem = (w0, w1)

    def one_pass(tab_hbm, idx, out_hbm):
        def gath(ch, slot):
            return pltpu.async_copy(
                tab_hbm.at[idx.at[ch]], rows.at[slot], gsem[slot])

        def wout(ch, slot):
            return pltpu.async_copy(
                rows.at[slot], out_hbm.at[pl.ds(base0 + ch * GCH, GCH)],
                wsem[slot])

        def wait_g(slot):
            pltpu.make_async_copy(
                tab_hbm.at[idx.at[0]], rows.at[slot], gsem[slot]).wait()

        def wait_w(slot):
            pltpu.make_async_copy(
                rows.at[slot], out_hbm.at[pl.ds(base0, GCH)],
                wsem[slot]).wait()

        gath(0, 0)

        @pl.loop(0, NPAIR)
        def _pair(ii):
            i0 = 2 * ii

            @pl.when(ii > 0)
            def _():
                wait_w(1)

            gath(i0 + 1, 1)
            wait_g(0)
            wout(i0, 0)
            wait_w(0)
            gath(i0 + 2, 0)
            wait_g(1)
            wout(i0 + 1, 1)

        wait_w(1)
        wait_g(0)
        wout(NGCH - 1, 0)
        wait_w(0)

    one_pass(p_hbm, sidx, gs_hbm)
    one_pass(q_hbm, didx, gd_hbm)


@functools.partial(
    pl.kernel,
    out_type=jax.ShapeDtypeStruct((NC_ * N, YW), jnp.float32),
    mesh=_mesh,
    scratch_types=(
        pltpu.VMEM((NIDX, CH), jnp.int32),
        pltpu.VMEM((2, G, CH, YW), jnp.float32),
        pltpu.VMEM_SHARED((N, YW), jnp.float32),
        pltpu.SemaphoreType.DMA,
        pltpu.SemaphoreType.DMA,
        pltpu.SemaphoreType.DMA,
        pltpu.SemaphoreType.DMA,
    ),
    compiler_params=_sc_params,
)
def _sc_scatter(y_hbm, dst_hbm, zer_hbm, z_hbm,
                didx, ybuf, table, l0, l1, a0, a1):
    c = lax.axis_index("c")
    s = lax.axis_index("s")
    w = c * NS_ + s
    row0 = w * NIDX
    stripe = pl.ds(s * NSTRIPE, NSTRIPE)
    pltpu.sync_copy(dst_hbm.at[pl.ds(row0, NIDX)], didx)
    pltpu.sync_copy(zer_hbm.at[stripe], table.at[stripe])

    lsem = (l0, l1)
    asem = (a0, a1)

    def load(ch, slot):
        return pltpu.async_copy(
            y_hbm.at[pl.ds(row0 + ch * G, G)], ybuf.at[slot], lsem[slot])

    def scat(ch, slot):
        for j in range(G):
            pltpu.async_copy(
                ybuf.at[slot].at[j], table.at[didx.at[ch * G + j]],
                asem[slot], add=True)

    def wait_l(slot):
        pltpu.make_async_copy(
            y_hbm.at[pl.ds(row0, G)], ybuf.at[slot], lsem[slot]).wait()

    def wait_a(slot):
        for j in range(G):
            pltpu.make_async_copy(
                ybuf.at[slot].at[j], table.at[didx.at[0]],
                asem[slot]).wait()

    load(0, 0)
    plsc.subcore_barrier()

    @pl.loop(0, NPAIR)
    def _pair(ii):
        i0 = 2 * ii

        @pl.when(ii > 0)
        def _():
            wait_a(1)

        load(i0 + 1, 1)
        wait_l(0)
        scat(i0, 0)
        wait_a(0)
        load(i0 + 2, 0)
        wait_l(1)
        scat(i0 + 1, 1)

    wait_a(1)
    wait_l(0)
    scat(NCHUNK - 1, 0)
    wait_a(0)

    plsc.subcore_barrier()
    pltpu.sync_copy(table.at[stripe], z_hbm.at[pl.ds(c * N + s * NSTRIPE,
                                                     NSTRIPE)])


def _matmul_call(f, n_rows, block_rows, n_in, extra_specs, out_shapes):
    grid = (n_rows // block_rows,)
    return pl.pallas_call(
        f,
        grid=grid,
        in_specs=[pl.BlockSpec((block_rows, n_in), lambda i: (i, 0))] + extra_specs,
        out_specs=[pl.BlockSpec((block_rows, s[1]), lambda i: (i, 0))
                   for s in out_shapes],
        out_shape=[jax.ShapeDtypeStruct(s, jnp.float32) for s in out_shapes],
    )


def _full(shape):
    return pl.BlockSpec(shape, lambda i: (0, 0))


BN = 1000
BE = 4000


def _node_pre_k(nf, ws, wd, wn, bn, pn_o, qn_o, nc_o):
    x = nf[...]
    pn_o[...] = jnp.dot(x, ws[...], preferred_element_type=jnp.float32)
    qn_o[...] = jnp.dot(x, wd[...], preferred_element_type=jnp.float32)
    nc_o[...] = jnp.dot(x, wn[...], preferred_element_type=jnp.float32) + bn[...]


def _edge_pre_k(ef, we, be, ec_o):
    ec_o[...] = jnp.dot(ef[...], we[...], preferred_element_type=jnp.float32) + be[...]


def _edge_k(gs, gd, hef_in, ec, wa, ba, we, hef_o, y_o):
    r = jnp.dot(hef_in[...], we[...], preferred_element_type=jnp.float32) + ec[...]
    hef = jnp.maximum(gs[...] + gd[...] + r, 0.0)
    hef_o[...] = hef
    logit = jnp.sum(hef * wa[...], axis=1, keepdims=True) + ba[...]
    ex = jnp.exp(logit)
    y_o[...] = jnp.concatenate(
        [hef * ex, ex, jnp.zeros((hef.shape[0], YW - HEF - 1), jnp.float32)],
        axis=1)


def _node_k(z0, z1, hnf, nc, wnh, wna, ws, wd, pn, qn, hnf_o, p_o, q_o):
    z = z0[...] + z1[...]
    agg = z[:, :HEF] / (z[:, HEF:HEF + 1] + 1e-16)
    h = (jnp.dot(hnf[...], wnh[...], preferred_element_type=jnp.float32)
         + jnp.dot(agg, wna[...], preferred_element_type=jnp.float32)
         + nc[...])
    h = jnp.maximum(h, 0.0)
    hnf_o[...] = h
    p_o[...] = jnp.dot(h, ws[...], preferred_element_type=jnp.float32) + pn[...]
    q_o[...] = jnp.dot(h, wd[...], preferred_element_type=jnp.float32) + qn[...]


def _readout_k(x, w, b, o):
    o[...] = jnp.dot(x[...], w[...], preferred_element_type=jnp.float32) + b[...]


def kernel(nf, ef, edge_index, n_iters, W_e, b_e, W_a, b_a, W_n, b_n,
           W_no, b_no, W_eo, b_eo):
    f32 = jnp.float32
    src4 = edge_index[0].reshape(E // GCH, GCH)
    dst4 = edge_index[1].reshape(E // GCH, GCH)
    dst2 = edge_index[1].reshape(E // CH, CH)

    We_s1 = W_e[0:128]
    We_s2 = W_e[128:256]
    We_d1 = W_e[256:384]
    We_d2 = W_e[384:512]
    We_e1 = W_e[512:576]
    We_e2 = W_e[576:592]
    Wn_h = W_n[0:128]
    Wn_nf = W_n[128:256]
    Wn_a = W_n[256:320]
    be = b_e.reshape(1, HEF)
    bn = b_n.reshape(1, HNF)
    wa = W_a.reshape(1, HEF)
    ba = b_a.reshape(1, 1)
    bno = b_no.reshape(1, 128)
    beo = b_eo.reshape(1, HEF)

    pn, qn, nc = _matmul_call(
        _node_pre_k, N, BN, NF_DIM,
        [_full((NF_DIM, HEF)), _full((NF_DIM, HEF)), _full((NF_DIM, HNF)),
         _full((1, HNF))],
        [(N, HEF), (N, HEF), (N, HNF)],
    )(nf, We_s2, We_d2, Wn_nf, bn)

    (ec,) = _matmul_call(
        _edge_pre_k, E, BE, 16,
        [_full((16, HEF)), _full((1, HEF))],
        [(E, HEF)],
    )(ef, We_e2, be)

    zer = jnp.zeros((N, YW), f32)
    hnf0 = jnp.zeros((N, HNF), f32)
    hef0 = jnp.zeros((E, HEF), f32)

    edge_call = _matmul_call(
        _edge_k, E, BE, HEF,
        [pl.BlockSpec((BE, HEF), lambda i: (i, 0)),
         pl.BlockSpec((BE, HEF), lambda i: (i, 0)),
         pl.BlockSpec((BE, HEF), lambda i: (i, 0)),
         _full((1, HEF)), _full((1, 1)), _full((HEF, HEF))],
        [(E, HEF), (E, YW)],
    )

    node_call = _matmul_call(
        _node_k, N, BN, YW,
        [pl.BlockSpec((BN, YW), lambda i: (i + N // BN, 0)),
         pl.BlockSpec((BN, HNF), lambda i: (i, 0)),
         pl.BlockSpec((BN, HNF), lambda i: (i, 0)),
         _full((HNF, HNF)), _full((HEF, HNF)),
         _full((HNF, HEF)), _full((HNF, HEF)),
         pl.BlockSpec((BN, HEF), lambda i: (i, 0)),
         pl.BlockSpec((BN, HEF), lambda i: (i, 0))],
        [(N, HNF), (N, HEF), (N, HEF)],
    )

    def body(_, carry):
        hnf, hef, p, q = carry
        gs, gd = _sc_gather(p, q, src4, dst4)
        hef2, y = edge_call(gs, gd, hef, ec, wa, ba, We_e1)
        z = _sc_scatter(y.reshape(E // CH, CH, YW), dst2, zer)
        hnf2, p2, q2 = node_call(z, z, hnf, nc, Wn_h, Wn_a,
                                 We_s1, We_d1, pn, qn)
        return (hnf2, hef2, p2, q2)

    hnf, hef, _, _ = lax.fori_loop(
        0, n_iters, body, (hnf0, hef0, pn, qn))

    (unf,) = _matmul_call(
        _readout_k, N, BN, HNF,
        [_full((HNF, 128)), _full((1, 128))],
        [(N, 128)],
    )(hnf, W_no, bno)

    (uef,) = _matmul_call(
        _readout_k, E, BE, HEF,
        [_full((HEF, HEF)), _full((1, HEF))],
        [(E, HEF)],
    )(hef, W_eo, beo)

    return (unf, uef)


# speedup vs baseline: 6.2586x; 1.2538x over previous
import functools

import jax
import jax.numpy as jnp
from jax import lax
from jax.experimental import pallas as pl
from jax.experimental.pallas import tpu as pltpu
from jax.experimental.pallas import tpu_sc as plsc

N = 10000
E = 320000
NF_DIM = 128
HNF = 128
HEF = 64
YW = 128

NC_ = 2
NS_ = 16
NW = NC_ * NS_
EW = E // NW
GCH = 400
NGCH = EW // GCH
NGPAIR = (NGCH - 1) // 2
SCH = 80
SG = 1
NSCH = EW // (SCH * SG)
NSPAIR = (NSCH - 1) // 2
NSTRIPE = 624
NTAIL = N - NS_ * NSTRIPE


def _m8(x):
    return pl.multiple_of(x, 8)

_mesh = plsc.VectorSubcoreMesh(core_axis_name="c", subcore_axis_name="s")


@functools.partial(
    pl.kernel,
    out_type=(
        jax.ShapeDtypeStruct((E, 128), jnp.float32),
        jax.ShapeDtypeStruct((E, 128), jnp.float32),
    ),
    mesh=_mesh,
    scratch_types=(
        pltpu.VMEM((EW,), jnp.int32),
        pltpu.VMEM((EW,), jnp.int32),
        pltpu.VMEM((2, GCH, 128), jnp.float32),
        pltpu.SemaphoreType.DMA,
        pltpu.SemaphoreType.DMA,
        pltpu.SemaphoreType.DMA,
        pltpu.SemaphoreType.DMA,
    ),
)
def _sc_gather(pq_hbm, src_hbm, dst_hbm, gs_hbm, gd_hbm,
               sidx, didx, rows, g0, g1, w0, w1):
    c = lax.axis_index("c")
    s = lax.axis_index("s")
    w = c * NS_ + s
    base0 = w * EW
    pltpu.sync_copy(src_hbm.at[pl.ds(base0, EW)], sidx)
    pltpu.sync_copy(dst_hbm.at[pl.ds(base0, EW)], didx)

    gsem = (g0, g1)
    wsem = (w0, w1)

    def one_pass(idx, out_hbm):
        def gath(ch, slot):
            return pltpu.async_copy(
                pq_hbm.at[idx.at[pl.ds(ch * GCH, GCH)]], rows.at[slot],
                gsem[slot])

        def wout(ch, slot):
            return pltpu.async_copy(
                rows.at[slot], out_hbm.at[pl.ds(_m8(base0 + ch * GCH), GCH)],
                wsem[slot])

        def wait_g(slot):
            pltpu.make_async_copy(
                pq_hbm.at[idx.at[pl.ds(0, GCH)]], rows.at[slot],
                gsem[slot]).wait()

        def wait_w(slot):
            pltpu.make_async_copy(
                rows.at[slot], out_hbm.at[pl.ds(base0, GCH)],
                wsem[slot]).wait()

        gath(0, 0)

        @pl.loop(0, NGPAIR)
        def _pair(ii):
            i0 = 2 * ii

            @pl.when(ii > 0)
            def _():
                wait_w(1)

            gath(i0 + 1, 1)
            wait_g(0)
            wout(i0, 0)
            wait_w(0)
            gath(i0 + 2, 0)
            wait_g(1)
            wout(i0 + 1, 1)

        wait_w(1)
        wait_g(0)
        wout(NGCH - 1, 0)
        wait_w(0)

    one_pass(sidx, gs_hbm)
    one_pass(didx, gd_hbm)


@functools.partial(
    pl.kernel,
    out_type=jax.ShapeDtypeStruct((NC_ * N, YW), jnp.float32),
    mesh=_mesh,
    scratch_types=(
        pltpu.VMEM((2, SCH, YW), jnp.float32),
        [pltpu.VMEM((SCH,), jnp.int32) for _ in range(2)],
        pltpu.VMEM_SHARED((N, YW), jnp.float32),
        pltpu.SemaphoreType.DMA,
        pltpu.SemaphoreType.DMA,
        pltpu.SemaphoreType.DMA,
        pltpu.SemaphoreType.DMA,
        pltpu.SemaphoreType.DMA,
        pltpu.SemaphoreType.DMA,
    ),
)
def _sc_scatter(y_hbm, dst_hbm, zer_hbm, z_hbm,
                ybuf, idxb, table, l0, l1, a0, a1, x0, x1):
    c = lax.axis_index("c")
    s = lax.axis_index("s")
    w = c * NS_ + s
    base0 = w * EW
    stripe = pl.ds(_m8(s * NSTRIPE), NSTRIPE)
    pltpu.sync_copy(zer_hbm.at[stripe], table.at[stripe])

    @pl.when(s == 0)
    def _ztail():
        tail = pl.ds(NS_ * NSTRIPE, NTAIL)
        pltpu.sync_copy(zer_hbm.at[tail], table.at[tail])

    lsem = (l0, l1)
    asem = (a0, a1)
    xsem = (x0, x1)

    def load(ch, slot):
        pltpu.async_copy(
            y_hbm.at[pl.ds(_m8(base0 + ch * SCH), SCH)],
            ybuf.at[slot], lsem[slot])
        pltpu.async_copy(
            dst_hbm.at[pl.ds(_m8(base0 + ch * SCH), SCH)],
            idxb[slot], xsem[slot])

    def scat(ch, slot):
        pltpu.async_copy(
            ybuf.at[slot], table.at[idxb[slot]], asem[slot], add=True)

    def wait_l(slot):
        pltpu.make_async_copy(
            y_hbm.at[pl.ds(base0, SCH)], ybuf.at[slot], lsem[slot]).wait()
        pltpu.make_async_copy(
            dst_hbm.at[pl.ds(base0, SCH)], idxb[slot], xsem[slot]).wait()

    def wait_a(slot):
        pltpu.make_async_copy(
            ybuf.at[slot], table.at[idxb[slot]], asem[slot]).wait()

    load(0, 0)
    plsc.subcore_barrier()

    @pl.loop(0, NSPAIR)
    def _pair(ii):
        i0 = 2 * ii

        @pl.when(ii > 0)
        def _():
            wait_a(1)

        load(i0 + 1, 1)
        wait_l(0)
        scat(i0, 0)
        wait_a(0)
        load(i0 + 2, 0)
        wait_l(1)
        scat(i0 + 1, 1)

    wait_a(1)
    wait_l(0)
    scat(NSCH - 1, 0)
    wait_a(0)

    plsc.subcore_barrier()
    pltpu.sync_copy(table.at[stripe],
                    z_hbm.at[pl.ds(_m8(c * N + s * NSTRIPE), NSTRIPE)])

    @pl.when(s == 0)
    def _wtail():
        tail = pl.ds(NS_ * NSTRIPE, NTAIL)
        pltpu.sync_copy(table.at[tail],
                        z_hbm.at[pl.ds(_m8(c * N + NS_ * NSTRIPE), NTAIL)])


def _matmul_call(f, n_rows, block_rows, n_in, extra_specs, out_shapes):
    grid = (n_rows // block_rows,)
    return pl.pallas_call(
        f,
        grid=grid,
        in_specs=[pl.BlockSpec((block_rows, n_in), lambda i: (i, 0))] + extra_specs,
        out_specs=[pl.BlockSpec((block_rows, s[1]), lambda i: (i, 0))
                   for s in out_shapes],
        out_shape=[jax.ShapeDtypeStruct(s, jnp.float32) for s in out_shapes],
    )


def _full(shape):
    return pl.BlockSpec(shape, lambda i: (0, 0))


BN = 1000
BE = 4000


def _node_pre_k(nf, wsd, wn, bn, pqn_o, nc_o):
    x = nf[...]
    pqn_o[...] = jnp.dot(x, wsd[...], preferred_element_type=jnp.float32)
    nc_o[...] = jnp.dot(x, wn[...], preferred_element_type=jnp.float32) + bn[...]


def _edge_pre_k(ef, we, be, ec_o):
    ec_o[...] = jnp.dot(ef[...], we[...], preferred_element_type=jnp.float32) + be[...]


def _edge_k(gs, gd, hef_in, ec, wa, ba, we, hef_o, y_o):
    r = jnp.dot(hef_in[...], we[...], preferred_element_type=jnp.float32) + ec[...]
    hef = jnp.maximum(gs[:, :HEF] + gd[:, HEF:] + r, 0.0)
    hef_o[...] = hef
    logit = jnp.sum(hef * wa[...], axis=1, keepdims=True) + ba[...]
    ex = jnp.exp(logit)
    y_o[...] = jnp.concatenate(
        [hef * ex, ex, jnp.zeros((hef.shape[0], YW - HEF - 1), jnp.float32)],
        axis=1)


def _node_k(z0, z1, hnf, nc, wnh, wna, wsd, pqn, hnf_o, pq_o):
    z = z0[...] + z1[...]
    agg = z[:, :HEF] / (z[:, HEF:HEF + 1] + 1e-16)
    h = (jnp.dot(hnf[...], wnh[...], preferred_element_type=jnp.float32)
         + jnp.dot(agg, wna[...], preferred_element_type=jnp.float32)
         + nc[...])
    h = jnp.maximum(h, 0.0)
    hnf_o[...] = h
    pq_o[...] = jnp.dot(h, wsd[...], preferred_element_type=jnp.float32) + pqn[...]


def _readout_k(x, w, b, o):
    o[...] = jnp.dot(x[...], w[...], preferred_element_type=jnp.float32) + b[...]


def kernel(nf, ef, edge_index, n_iters, W_e, b_e, W_a, b_a, W_n, b_n,
           W_no, b_no, W_eo, b_eo):
    f32 = jnp.float32
    src = edge_index[0]
    dst = edge_index[1]

    We_sd1 = jnp.concatenate([W_e[0:128], W_e[256:384]], axis=1)
    We_sd2 = jnp.concatenate([W_e[128:256], W_e[384:512]], axis=1)
    We_e1 = W_e[512:576]
    We_e2 = W_e[576:592]
    Wn_h = W_n[0:128]
    Wn_nf = W_n[128:256]
    Wn_a = W_n[256:320]
    be = b_e.reshape(1, HEF)
    bn = b_n.reshape(1, HNF)
    wa = W_a.reshape(1, HEF)
    ba = b_a.reshape(1, 1)
    bno = b_no.reshape(1, 128)
    beo = b_eo.reshape(1, HEF)

    pqn, nc = _matmul_call(
        _node_pre_k, N, BN, NF_DIM,
        [_full((NF_DIM, 128)), _full((NF_DIM, HNF)), _full((1, HNF))],
        [(N, 128), (N, HNF)],
    )(nf, We_sd2, Wn_nf, bn)

    (ec,) = _matmul_call(
        _edge_pre_k, E, BE, 16,
        [_full((16, HEF)), _full((1, HEF))],
        [(E, HEF)],
    )(ef, We_e2, be)

    zer = jnp.zeros((N, YW), f32)
    hnf0 = jnp.zeros((N, HNF), f32)
    hef0 = jnp.zeros((E, HEF), f32)

    edge_call = _matmul_call(
        _edge_k, E, BE, 128,
        [pl.BlockSpec((BE, 128), lambda i: (i, 0)),
         pl.BlockSpec((BE, HEF), lambda i: (i, 0)),
         pl.BlockSpec((BE, HEF), lambda i: (i, 0)),
         _full((1, HEF)), _full((1, 1)), _full((HEF, HEF))],
        [(E, HEF), (E, YW)],
    )

    node_call = _matmul_call(
        _node_k, N, BN, YW,
        [pl.BlockSpec((BN, YW), lambda i: (i + N // BN, 0)),
         pl.BlockSpec((BN, HNF), lambda i: (i, 0)),
         pl.BlockSpec((BN, HNF), lambda i: (i, 0)),
         _full((HNF, HNF)), _full((HEF, HNF)), _full((HNF, 128)),
         pl.BlockSpec((BN, 128), lambda i: (i, 0))],
        [(N, HNF), (N, 128)],
    )

    def body(_, carry):
        hnf, hef, pq = carry
        gs, gd = _sc_gather(pq, src, dst)
        hef2, y = edge_call(gs, gd, hef, ec, wa, ba, We_e1)
        z = _sc_scatter(y, dst, zer)
        hnf2, pq2 = node_call(z, z, hnf, nc, Wn_h, Wn_a, We_sd1, pqn)
        return (hnf2, hef2, pq2)

    hnf, hef, _ = lax.fori_loop(0, n_iters, body, (hnf0, hef0, pqn))

    (unf,) = _matmul_call(
        _readout_k, N, BN, HNF,
        [_full((HNF, 128)), _full((1, 128))],
        [(N, 128)],
    )(hnf, W_no, bno)

    (uef,) = _matmul_call(
        _readout_k, E, BE, HEF,
        [_full((HEF, HEF)), _full((1, HEF))],
        [(E, HEF)],
    )(hef, W_eo, beo)

    return (unf, uef)


# speedup vs baseline: 6.9783x; 1.1150x over previous
import functools

import jax
import jax.numpy as jnp
from jax import lax
from jax.experimental import pallas as pl
from jax.experimental.pallas import tpu as pltpu
from jax.experimental.pallas import tpu_sc as plsc

N = 10000
E = 320000
NF_DIM = 128
HNF = 128
HEF = 64
YW = 128

NC_ = 2
NS_ = 16
NW = NC_ * NS_
EW = E // NW
GCH = 400
NGCH = EW // GCH
NGPAIR = (NGCH - 1) // 2
SCH = 80
SG = 1
NSCH = EW // (SCH * SG)
NSPAIR = (NSCH - 1) // 2
NSTRIPE = 624
NTAIL = N - NS_ * NSTRIPE


def _m8(x):
    return pl.multiple_of(x, 8)


def _m16(x):
    return pl.multiple_of(x, 16)

_mesh = plsc.VectorSubcoreMesh(core_axis_name="c", subcore_axis_name="s")


@functools.partial(
    pl.kernel,
    out_type=(
        jax.ShapeDtypeStruct((E, 128), jnp.float32),
        jax.ShapeDtypeStruct((E, 128), jnp.float32),
    ),
    mesh=_mesh,
    scratch_types=(
        pltpu.VMEM((EW,), jnp.int32),
        pltpu.VMEM((EW,), jnp.int32),
        pltpu.VMEM((2, GCH, 128), jnp.float32),
        pltpu.SemaphoreType.DMA,
        pltpu.SemaphoreType.DMA,
        pltpu.SemaphoreType.DMA,
        pltpu.SemaphoreType.DMA,
    ),
)
def _sc_gather(pq_hbm, src_hbm, dst_hbm, gs_hbm, gd_hbm,
               sidx, didx, rows, g0, g1, w0, w1):
    c = lax.axis_index("c")
    s = lax.axis_index("s")
    w = c * NS_ + s
    base0 = w * EW
    pltpu.sync_copy(src_hbm.at[pl.ds(base0, EW)], sidx)
    pltpu.sync_copy(dst_hbm.at[pl.ds(base0, EW)], didx)

    gsem = (g0, g1)
    wsem = (w0, w1)

    def one_pass(idx, out_hbm):
        def gath(ch, slot):
            return pltpu.async_copy(
                pq_hbm.at[idx.at[pl.ds(ch * GCH, GCH)]], rows.at[slot],
                gsem[slot])

        def wout(ch, slot):
            return pltpu.async_copy(
                rows.at[slot], out_hbm.at[pl.ds(_m8(base0 + ch * GCH), GCH)],
                wsem[slot])

        def wait_g(slot):
            pltpu.make_async_copy(
                pq_hbm.at[idx.at[pl.ds(0, GCH)]], rows.at[slot],
                gsem[slot]).wait()

        def wait_w(slot):
            pltpu.make_async_copy(
                rows.at[slot], out_hbm.at[pl.ds(base0, GCH)],
                wsem[slot]).wait()

        gath(0, 0)

        @pl.loop(0, NGPAIR)
        def _pair(ii):
            i0 = 2 * ii

            @pl.when(ii > 0)
            def _():
                wait_w(1)

            gath(i0 + 1, 1)
            wait_g(0)
            wout(i0, 0)
            wait_w(0)
            gath(i0 + 2, 0)
            wait_g(1)
            wout(i0 + 1, 1)

        wait_w(1)
        wait_g(0)
        wout(NGCH - 1, 0)
        wait_w(0)

    one_pass(sidx, gs_hbm)
    one_pass(didx, gd_hbm)


@functools.partial(
    pl.kernel,
    out_type=jax.ShapeDtypeStruct((NC_ * N, YW), jnp.float32),
    mesh=_mesh,
    scratch_types=(
        pltpu.VMEM((2, SCH, YW), jnp.float32),
        [pltpu.VMEM((SCH,), jnp.int32) for _ in range(2)],
        pltpu.VMEM_SHARED((N, YW), jnp.float32),
        pltpu.SemaphoreType.DMA,
        pltpu.SemaphoreType.DMA,
        pltpu.SemaphoreType.DMA,
        pltpu.SemaphoreType.DMA,
        pltpu.SemaphoreType.DMA,
        pltpu.SemaphoreType.DMA,
    ),
)
def _sc_scatter(y_hbm, dst_hbm, zer_hbm, z_hbm,
                ybuf, idxb, table, l0, l1, a0, a1, x0, x1):
    c = lax.axis_index("c")
    s = lax.axis_index("s")
    w = c * NS_ + s
    base0 = w * EW
    stripe = pl.ds(_m8(s * NSTRIPE), NSTRIPE)
    pltpu.sync_copy(zer_hbm.at[stripe], table.at[stripe])

    @pl.when(s == 0)
    def _ztail():
        tail = pl.ds(NS_ * NSTRIPE, NTAIL)
        pltpu.sync_copy(zer_hbm.at[tail], table.at[tail])

    lsem = (l0, l1)
    asem = (a0, a1)
    xsem = (x0, x1)

    def load(ch, slot):
        pltpu.async_copy(
            y_hbm.at[pl.ds(_m8(base0 + ch * SCH), SCH)],
            ybuf.at[slot], lsem[slot])
        pltpu.async_copy(
            dst_hbm.at[pl.ds(_m8(base0 + ch * SCH), SCH)],
            idxb[slot], xsem[slot])

    def scat(ch, slot):
        pltpu.async_copy(
            ybuf.at[slot], table.at[idxb[slot]], asem[slot], add=True)

    def wait_l(slot):
        pltpu.make_async_copy(
            y_hbm.at[pl.ds(base0, SCH)], ybuf.at[slot], lsem[slot]).wait()
        pltpu.make_async_copy(
            dst_hbm.at[pl.ds(base0, SCH)], idxb[slot], xsem[slot]).wait()

    def wait_a(slot):
        pltpu.make_async_copy(
            ybuf.at[slot], table.at[idxb[slot]], asem[slot]).wait()

    load(0, 0)
    plsc.subcore_barrier()

    @pl.loop(0, NSPAIR)
    def _pair(ii):
        i0 = 2 * ii

        @pl.when(ii > 0)
        def _():
            wait_a(1)

        load(i0 + 1, 1)
        wait_l(0)
        scat(i0, 0)
        wait_a(0)
        load(i0 + 2, 0)
        wait_l(1)
        scat(i0 + 1, 1)

    wait_a(1)
    wait_l(0)
    scat(NSCH - 1, 0)
    wait_a(0)

    plsc.subcore_barrier()
    pltpu.sync_copy(table.at[stripe],
                    z_hbm.at[pl.ds(_m8(c * N + s * NSTRIPE), NSTRIPE)])

    @pl.when(s == 0)
    def _wtail():
        tail = pl.ds(NS_ * NSTRIPE, NTAIL)
        pltpu.sync_copy(table.at[tail],
                        z_hbm.at[pl.ds(_m8(c * N + NS_ * NSTRIPE), NTAIL)])


def _matmul_call(f, n_rows, block_rows, n_in, extra_specs, out_shapes):
    grid = (n_rows // block_rows,)
    return pl.pallas_call(
        f,
        grid=grid,
        in_specs=[pl.BlockSpec((block_rows, n_in), lambda i: (i, 0))] + extra_specs,
        out_specs=[pl.BlockSpec((block_rows, s.shape[1]), lambda i: (i, 0))
                   for s in out_shapes],
        out_shape=list(out_shapes),
    )


def _sds(shape, dtype=jnp.float32):
    return jax.ShapeDtypeStruct(shape, dtype)


def _full(shape):
    return pl.BlockSpec(shape, lambda i: (0, 0))


BN = 1000
BE = 4000


def _node_pre_k(nf, wsd, wn, bn, pqn_o, nc_o):
    x = nf[...]
    pqn_o[...] = jnp.dot(x, wsd[...], preferred_element_type=jnp.float32)
    nc_o[...] = jnp.dot(x, wn[...], preferred_element_type=jnp.float32) + bn[...]


def _edge_k(gs, gd, hef_in, ef, wa, ba, we, we2, be, hef_o, y_o):
    r = (jnp.dot(hef_in[...], we[...], preferred_element_type=jnp.float32)
         + jnp.dot(ef[...], we2[...], preferred_element_type=jnp.float32)
         + be[...])
    hef = jnp.maximum(gs[:, :HEF] + gd[:, HEF:] + r, 0.0)
    hef_o[...] = hef
    logit = jnp.sum(hef * wa[...], axis=1, keepdims=True) + ba[...]
    ex = jnp.exp(logit)
    y_o[...] = jnp.concatenate(
        [hef * ex, ex, jnp.zeros((hef.shape[0], YW - HEF - 1), jnp.float32)],
        axis=1)


def _node_k(z0, z1, hnf, nc, wnh, wna, wsd, pqn, hnf_o, pq_o):
    z = z0[...] + z1[...]
    agg = z[:, :HEF] / (z[:, HEF:HEF + 1] + 1e-16)
    h = (jnp.dot(hnf[...], wnh[...], preferred_element_type=jnp.float32)
         + jnp.dot(agg, wna[...], preferred_element_type=jnp.float32)
         + nc[...])
    h = jnp.maximum(h, 0.0)
    hnf_o[...] = h
    pq_o[...] = jnp.dot(h, wsd[...], preferred_element_type=jnp.float32) + pqn[...]


def _readout_k(x, w, b, o):
    o[...] = jnp.dot(x[...], w[...], preferred_element_type=jnp.float32) + b[...]


def kernel(nf, ef, edge_index, n_iters, W_e, b_e, W_a, b_a, W_n, b_n,
           W_no, b_no, W_eo, b_eo):
    f32 = jnp.float32
    src = edge_index[0]
    dst = edge_index[1]

    We_sd1 = jnp.concatenate([W_e[0:128], W_e[256:384]], axis=1)
    We_sd2 = jnp.concatenate([W_e[128:256], W_e[384:512]], axis=1)
    We_e1 = W_e[512:576]
    We_e2 = W_e[576:592]
    Wn_h = W_n[0:128]
    Wn_nf = W_n[128:256]
    Wn_a = W_n[256:320]
    be = b_e.reshape(1, HEF)
    bn = b_n.reshape(1, HNF)
    wa = W_a.reshape(1, HEF)
    ba = b_a.reshape(1, 1)
    bno = b_no.reshape(1, 128)
    beo = b_eo.reshape(1, HEF)

    pqn, nc = _matmul_call(
        _node_pre_k, N, BN, NF_DIM,
        [_full((NF_DIM, 128)), _full((NF_DIM, HNF)), _full((1, HNF))],
        [_sds((N, 128)), _sds((N, HNF))],
    )(nf, We_sd2, Wn_nf, bn)

    zer = jnp.zeros((N, YW), f32)
    hnf0 = jnp.zeros((N, HNF), f32)
    hef0 = jnp.zeros((E, HEF), f32)

    edge_call = pl.pallas_call(
        _edge_k,
        grid=(E // BE,),
        in_specs=[
            pl.BlockSpec((BE, 128), lambda i: (i, 0)),
            pl.BlockSpec((BE, 128), lambda i: (i, 0)),
            pl.BlockSpec((BE, HEF), lambda i: (i, 0)),
            pl.BlockSpec((BE, 16), lambda i: (i, 0)),
            _full((1, HEF)), _full((1, 1)), _full((HEF, HEF)),
            _full((16, HEF)), _full((1, HEF))],
        out_specs=[pl.BlockSpec((BE, HEF), lambda i: (i, 0)),
                   pl.BlockSpec((BE, YW), lambda i: (i, 0))],
        out_shape=[_sds((E, HEF)), _sds((E, YW))],
    )

    node_call = _matmul_call(
        _node_k, N, BN, YW,
        [pl.BlockSpec((BN, YW), lambda i: (i + N // BN, 0)),
         pl.BlockSpec((BN, HNF), lambda i: (i, 0)),
         pl.BlockSpec((BN, HNF), lambda i: (i, 0)),
         _full((HNF, HNF)), _full((HEF, HNF)), _full((HNF, 128)),
         pl.BlockSpec((BN, 128), lambda i: (i, 0))],
        [_sds((N, HNF)), _sds((N, 128))],
    )

    def body(_, carry):
        hnf, hef, pq = carry
        gs, gd = _sc_gather(pq, src, dst)
        hef2, y = edge_call(gs, gd, hef, ef, wa, ba, We_e1, We_e2, be)
        z = _sc_scatter(y, dst, zer)
        hnf2, pq2 = node_call(z, z, hnf, nc, Wn_h, Wn_a, We_sd1, pqn)
        return (hnf2, hef2, pq2)

    hnf, hef, _ = lax.fori_loop(0, n_iters, body, (hnf0, hef0, pqn))

    (unf,) = _matmul_call(
        _readout_k, N, BN, HNF,
        [_full((HNF, 128)), _full((1, 128))],
        [_sds((N, 128))],
    )(hnf, W_no, bno)

    (uef,) = _matmul_call(
        _readout_k, E, BE, HEF,
        [_full((HEF, HEF)), _full((1, HEF))],
        [_sds((E, HEF))],
    )(hef, W_eo, beo)

    return (unf, uef)


# speedup vs baseline: 7.4187x; 1.0631x over previous
import functools

import jax
import jax.numpy as jnp
from jax import lax
from jax.experimental import pallas as pl
from jax.experimental.pallas import tpu as pltpu
from jax.experimental.pallas import tpu_sc as plsc

N = 10000
E = 320000
NF_DIM = 128
HNF = 128
HEF = 64
YW = 128

NC_ = 2
NS_ = 16
NW = NC_ * NS_
EW = E // NW
GCH = 200
NGCH = EW // GCH
NGPAIR = (NGCH - 2) // 2
SCH = 80
SG = 1
NSCH = EW // (SCH * SG)
NSPAIR = (NSCH - 1) // 2
NSTRIPE = 624
NTAIL = N - NS_ * NSTRIPE


def _m8(x):
    return pl.multiple_of(x, 8)


def _m16(x):
    return pl.multiple_of(x, 16)

_mesh = plsc.VectorSubcoreMesh(core_axis_name="c", subcore_axis_name="s")


@functools.partial(
    pl.kernel,
    out_type=jax.ShapeDtypeStruct((E, 128), jnp.float32),
    mesh=_mesh,
    scratch_types=(
        pltpu.VMEM((EW,), jnp.int32),
        pltpu.VMEM((EW,), jnp.int32),
        pltpu.VMEM((2, GCH, 128), jnp.float32),
        pltpu.VMEM((2, GCH, 128), jnp.float32),
        pltpu.SemaphoreType.DMA,
        pltpu.SemaphoreType.DMA,
        pltpu.SemaphoreType.DMA,
        pltpu.SemaphoreType.DMA,
        pltpu.SemaphoreType.DMA,
        pltpu.SemaphoreType.DMA,
    ),
)
def _sc_gather(pq_hbm, src_hbm, dst_hbm, gsd_hbm,
               sidx, didx, rows_s, rows_d, gs0, gs1, gd0, gd1, w0, w1):
    c = lax.axis_index("c")
    s = lax.axis_index("s")
    w = c * NS_ + s
    base0 = w * EW
    pltpu.sync_copy(src_hbm.at[pl.ds(base0, EW)], sidx)
    pltpu.sync_copy(dst_hbm.at[pl.ds(base0, EW)], didx)

    gssem = (gs0, gs1)
    gdsem = (gd0, gd1)
    wsem = (w0, w1)

    def gath(ch, slot):
        pltpu.async_copy(
            pq_hbm.at[sidx.at[pl.ds(ch * GCH, GCH)]], rows_s.at[slot],
            gssem[slot])
        pltpu.async_copy(
            pq_hbm.at[didx.at[pl.ds(ch * GCH, GCH)]], rows_d.at[slot],
            gdsem[slot])

    def wout(ch, slot):
        return pltpu.async_copy(
            rows_s.at[slot], gsd_hbm.at[pl.ds(_m8(base0 + ch * GCH), GCH)],
            wsem[slot])

    def wait_g(slot):
        pltpu.make_async_copy(
            pq_hbm.at[sidx.at[pl.ds(0, GCH)]], rows_s.at[slot],
            gssem[slot]).wait()
        pltpu.make_async_copy(
            pq_hbm.at[didx.at[pl.ds(0, GCH)]], rows_d.at[slot],
            gdsem[slot]).wait()

    def wait_w(slot):
        pltpu.make_async_copy(
            rows_s.at[slot], gsd_hbm.at[pl.ds(base0, GCH)],
            wsem[slot]).wait()

    def add_halves(slot):
        @pl.loop(0, GCH, unroll=2)
        def _add(r):
            for k in range(HEF // 16):
                sl = pl.ds(k * 16, 16)
                sr = pl.ds(HEF + k * 16, 16)
                rows_s[slot, r, sl] = rows_s[slot, r, sl] + rows_d[slot, r, sr]

    gath(0, 0)

    @pl.loop(0, NGPAIR)
    def _pair(ii):
        i0 = 2 * ii

        @pl.when(ii > 0)
        def _():
            wait_w(1)

        gath(i0 + 1, 1)
        wait_g(0)
        add_halves(0)
        wout(i0, 0)
        wait_w(0)
        gath(i0 + 2, 0)
        wait_g(1)
        add_halves(1)
        wout(i0 + 1, 1)

    wait_w(1)
    gath(NGCH - 1, 1)
    wait_g(0)
    add_halves(0)
    wout(NGCH - 2, 0)
    wait_w(0)
    wait_g(1)
    add_halves(1)
    wout(NGCH - 1, 1)
    wait_w(1)


@functools.partial(
    pl.kernel,
    out_type=jax.ShapeDtypeStruct((NC_ * N, YW), jnp.float32),
    mesh=_mesh,
    scratch_types=(
        pltpu.VMEM((2, SCH, YW), jnp.float32),
        [pltpu.VMEM((SCH,), jnp.int32) for _ in range(2)],
        pltpu.VMEM_SHARED((N, YW), jnp.float32),
        pltpu.SemaphoreType.DMA,
        pltpu.SemaphoreType.DMA,
        pltpu.SemaphoreType.DMA,
        pltpu.SemaphoreType.DMA,
        pltpu.SemaphoreType.DMA,
        pltpu.SemaphoreType.DMA,
    ),
)
def _sc_scatter(y_hbm, dst_hbm, zer_hbm, z_hbm,
                ybuf, idxb, table, l0, l1, a0, a1, x0, x1):
    c = lax.axis_index("c")
    s = lax.axis_index("s")
    w = c * NS_ + s
    base0 = w * EW
    stripe = pl.ds(_m8(s * NSTRIPE), NSTRIPE)
    pltpu.sync_copy(zer_hbm.at[stripe], table.at[stripe])

    @pl.when(s == 0)
    def _ztail():
        tail = pl.ds(NS_ * NSTRIPE, NTAIL)
        pltpu.sync_copy(zer_hbm.at[tail], table.at[tail])

    lsem = (l0, l1)
    asem = (a0, a1)
    xsem = (x0, x1)

    def load(ch, slot):
        pltpu.async_copy(
            y_hbm.at[pl.ds(_m8(base0 + ch * SCH), SCH)],
            ybuf.at[slot], lsem[slot])
        pltpu.async_copy(
            dst_hbm.at[pl.ds(_m8(base0 + ch * SCH), SCH)],
            idxb[slot], xsem[slot])

    def scat(ch, slot):
        pltpu.async_copy(
            ybuf.at[slot], table.at[idxb[slot]], asem[slot], add=True)

    def wait_l(slot):
        pltpu.make_async_copy(
            y_hbm.at[pl.ds(base0, SCH)], ybuf.at[slot], lsem[slot]).wait()
        pltpu.make_async_copy(
            dst_hbm.at[pl.ds(base0, SCH)], idxb[slot], xsem[slot]).wait()

    def wait_a(slot):
        pltpu.make_async_copy(
            ybuf.at[slot], table.at[idxb[slot]], asem[slot]).wait()

    load(0, 0)
    plsc.subcore_barrier()

    @pl.loop(0, NSPAIR)
    def _pair(ii):
        i0 = 2 * ii

        @pl.when(ii > 0)
        def _():
            wait_a(1)

        load(i0 + 1, 1)
        wait_l(0)
        scat(i0, 0)
        wait_a(0)
        load(i0 + 2, 0)
        wait_l(1)
        scat(i0 + 1, 1)

    wait_a(1)
    wait_l(0)
    scat(NSCH - 1, 0)
    wait_a(0)

    plsc.subcore_barrier()
    pltpu.sync_copy(table.at[stripe],
                    z_hbm.at[pl.ds(_m8(c * N + s * NSTRIPE), NSTRIPE)])

    @pl.when(s == 0)
    def _wtail():
        tail = pl.ds(NS_ * NSTRIPE, NTAIL)
        pltpu.sync_copy(table.at[tail],
                        z_hbm.at[pl.ds(_m8(c * N + NS_ * NSTRIPE), NTAIL)])


def _matmul_call(f, n_rows, block_rows, n_in, extra_specs, out_shapes):
    grid = (n_rows // block_rows,)
    return pl.pallas_call(
        f,
        grid=grid,
        in_specs=[pl.BlockSpec((block_rows, n_in), lambda i: (i, 0))] + extra_specs,
        out_specs=[pl.BlockSpec((block_rows, s.shape[1]), lambda i: (i, 0))
                   for s in out_shapes],
        out_shape=list(out_shapes),
    )


def _sds(shape, dtype=jnp.float32):
    return jax.ShapeDtypeStruct(shape, dtype)


def _full(shape):
    return pl.BlockSpec(shape, lambda i: (0, 0))


BN = 1000
BE = 4000


def _node_pre_k(nf, wsd, wn, bn, pqn_o, nc_o):
    x = nf[...]
    pqn_o[...] = jnp.dot(x, wsd[...], preferred_element_type=jnp.float32)
    nc_o[...] = jnp.dot(x, wn[...], preferred_element_type=jnp.float32) + bn[...]


def _edge_k(gsd, hef_in, ef, wa, ba, we, we2, be, hef_o, y_o):
    r = (jnp.dot(hef_in[...], we[...], preferred_element_type=jnp.float32)
         + jnp.dot(ef[...], we2[...], preferred_element_type=jnp.float32)
         + be[...])
    hef = jnp.maximum(gsd[:, :HEF] + r, 0.0)
    hef_o[...] = hef
    logit = jnp.sum(hef * wa[...], axis=1, keepdims=True) + ba[...]
    ex = jnp.exp(logit)
    y_o[...] = jnp.concatenate(
        [hef * ex, ex, jnp.zeros((hef.shape[0], YW - HEF - 1), jnp.float32)],
        axis=1)


def _node_k(z0, z1, hnf, nc, wnh, wna, wsd, pqn, hnf_o, pq_o):
    z = z0[...] + z1[...]
    agg = z[:, :HEF] / (z[:, HEF:HEF + 1] + 1e-16)
    h = (jnp.dot(hnf[...], wnh[...], preferred_element_type=jnp.float32)
         + jnp.dot(agg, wna[...], preferred_element_type=jnp.float32)
         + nc[...])
    h = jnp.maximum(h, 0.0)
    hnf_o[...] = h
    pq_o[...] = jnp.dot(h, wsd[...], preferred_element_type=jnp.float32) + pqn[...]


def _readout_k(x, w, b, o):
    o[...] = jnp.dot(x[...], w[...], preferred_element_type=jnp.float32) + b[...]


def kernel(nf, ef, edge_index, n_iters, W_e, b_e, W_a, b_a, W_n, b_n,
           W_no, b_no, W_eo, b_eo):
    f32 = jnp.float32
    src = edge_index[0]
    dst = edge_index[1]

    We_sd1 = jnp.concatenate([W_e[0:128], W_e[256:384]], axis=1)
    We_sd2 = jnp.concatenate([W_e[128:256], W_e[384:512]], axis=1)
    We_e1 = W_e[512:576]
    We_e2 = W_e[576:592]
    Wn_h = W_n[0:128]
    Wn_nf = W_n[128:256]
    Wn_a = W_n[256:320]
    be = b_e.reshape(1, HEF)
    bn = b_n.reshape(1, HNF)
    wa = W_a.reshape(1, HEF)
    ba = b_a.reshape(1, 1)
    bno = b_no.reshape(1, 128)
    beo = b_eo.reshape(1, HEF)

    pqn, nc = _matmul_call(
        _node_pre_k, N, BN, NF_DIM,
        [_full((NF_DIM, 128)), _full((NF_DIM, HNF)), _full((1, HNF))],
        [_sds((N, 128)), _sds((N, HNF))],
    )(nf, We_sd2, Wn_nf, bn)

    zer = jnp.zeros((N, YW), f32)
    hnf0 = jnp.zeros((N, HNF), f32)
    hef0 = jnp.zeros((E, HEF), f32)

    edge_call = pl.pallas_call(
        _edge_k,
        grid=(E // BE,),
        in_specs=[
            pl.BlockSpec((BE, 128), lambda i: (i, 0)),
            pl.BlockSpec((BE, HEF), lambda i: (i, 0)),
            pl.BlockSpec((BE, 16), lambda i: (i, 0)),
            _full((1, HEF)), _full((1, 1)), _full((HEF, HEF)),
            _full((16, HEF)), _full((1, HEF))],
        out_specs=[pl.BlockSpec((BE, HEF), lambda i: (i, 0)),
                   pl.BlockSpec((BE, YW), lambda i: (i, 0))],
        out_shape=[_sds((E, HEF)), _sds((E, YW))],
    )

    node_call = _matmul_call(
        _node_k, N, BN, YW,
        [pl.BlockSpec((BN, YW), lambda i: (i + N // BN, 0)),
         pl.BlockSpec((BN, HNF), lambda i: (i, 0)),
         pl.BlockSpec((BN, HNF), lambda i: (i, 0)),
         _full((HNF, HNF)), _full((HEF, HNF)), _full((HNF, 128)),
         pl.BlockSpec((BN, 128), lambda i: (i, 0))],
        [_sds((N, HNF)), _sds((N, 128))],
    )

    def body(_, carry):
        hnf, hef, pq = carry
        gsd = _sc_gather(pq, src, dst)
        hef2, y = edge_call(gsd, hef, ef, wa, ba, We_e1, We_e2, be)
        z = _sc_scatter(y, dst, zer)
        hnf2, pq2 = node_call(z, z, hnf, nc, Wn_h, Wn_a, We_sd1, pqn)
        return (hnf2, hef2, pq2)

    hnf, hef, _ = lax.fori_loop(0, n_iters, body, (hnf0, hef0, pqn))

    (unf,) = _matmul_call(
        _readout_k, N, BN, HNF,
        [_full((HNF, 128)), _full((1, 128))],
        [_sds((N, 128))],
    )(hnf, W_no, bno)

    (uef,) = _matmul_call(
        _readout_k, E, BE, HEF,
        [_full((HEF, HEF)), _full((1, HEF))],
        [_sds((E, HEF))],
    )(hef, W_eo, beo)

    return (unf, uef)
